# baseline (speedup 1.0000x reference)
import functools

import jax
import jax.numpy as jnp
from jax import lax
from jax.experimental import pallas as pl
from jax.experimental.pallas import tpu as pltpu
from jax.experimental.pallas import tpu_sc as plsc

N_NODES = 10000
IN_CH = 128
D = 64
AW = 72
HSW = 40
NP = 10240
EB = 128
NBLK = 82
WPE = EB * NBLK
NW = 32
EP = WPE * NW
RPT = NP // 16

_PERM = [32 * j + 2 * m + o for j in (0, 1) for o in (0, 1) for m in range(16)]


def _leaky(v):
    return jnp.maximum(v, 0.2 * v)


def _tc_pre_body(x_ref, w_ref, ams_ref, amd_ref, h_ref, as_ref, ad_ref, m_ref):
    h = jnp.dot(x_ref[...], w_ref[...], preferred_element_type=jnp.float32)
    h_ref[...] = h
    a_s = jnp.dot(h, ams_ref[...], preferred_element_type=jnp.float32)
    a_d = jnp.dot(h, amd_ref[...], preferred_element_type=jnp.float32)
    as_ref[...] = a_s
    ad_ref[...] = a_d
    m_ref[...] = _leaky(a_s.max(axis=0) + a_d.max(axis=0)).reshape(1, 16)


def _tc_mid_body(a0_ref, a1_ref, b_ref, rep_ref, w_ref,
                 ams_ref, amd_ref, h_ref, as_ref, ad_ref, m_ref):
    s = a0_ref[:, 0:D] + a1_ref[:, 0:D]
    dp = a0_ref[:, D:D + 8] + a1_ref[:, D:D + 8]
    d64 = jnp.dot(dp, rep_ref[...], preferred_element_type=jnp.float32) + 1e-16
    x2 = s / d64 + b_ref[...]
    x2 = jnp.where(x2 > 0, x2, jnp.exp(jnp.minimum(x2, 0.0)) - 1.0)
    h = jnp.dot(x2, w_ref[...], preferred_element_type=jnp.float32)
    h_ref[...] = h
    a_s = jnp.dot(h, ams_ref[...], preferred_element_type=jnp.float32)
    a_d = jnp.dot(h, amd_ref[...], preferred_element_type=jnp.float32)
    as_ref[...] = a_s
    ad_ref[...] = a_d
    m_ref[...] = _leaky(a_s.max(axis=0) + a_d.max(axis=0)).reshape(1, 16)


def _tc_post_body(a0_ref, a1_ref, b_ref, up_ref, o_ref):
    s = a0_ref[:, 0:D] + a1_ref[:, 0:D]
    dp = a0_ref[:, D:D + 1] + a1_ref[:, D:D + 1]
    o = s / (dp + 1e-16) + b_ref[...]
    z = o - jnp.max(o, axis=1, keepdims=True)
    z = z - jnp.log(jnp.sum(jnp.exp(z), axis=1, keepdims=True))
    o_ref[...] = jnp.dot(z, up_ref[...], preferred_element_type=jnp.float32)


def _tc_pre(xp, W, ams16, amd16):
    return pl.pallas_call(
        _tc_pre_body,
        out_shape=(
            jax.ShapeDtypeStruct((NP, D), jnp.float32),
            jax.ShapeDtypeStruct((NP, 16), jnp.float32),
            jax.ShapeDtypeStruct((NP, 16), jnp.float32),
            jax.ShapeDtypeStruct((1, 16), jnp.float32),
        ),
    )(xp, W, ams16, amd16)


def _tc_mid(a0, a1, b, rep, W, ams16, amd16):
    return pl.pallas_call(
        _tc_mid_body,
        out_shape=(
            jax.ShapeDtypeStruct((NP, D), jnp.float32),
            jax.ShapeDtypeStruct((NP, 16), jnp.float32),
            jax.ShapeDtypeStruct((NP, 16), jnp.float32),
            jax.ShapeDtypeStruct((1, 16), jnp.float32),
        ),
    )(a0, a1, b, rep, W, ams16, amd16)


def _tc_post(a0, a1, b, up):
    return pl.pallas_call(
        _tc_post_body,
        out_shape=jax.ShapeDtypeStruct((NP, D), jnp.float32),
    )(a0, a1, b, up)


def _sc_body(hs_hbm, ad_hbm, m_hbm, src_hbm, dst_hbm, acc_hbm,
             sidx, didx, adt, hsr, mb, mv, zb, acc_s, gsem, ssem):
    core = lax.axis_index("c")
    sub = lax.axis_index("s")
    wid = sub * 2 + core
    zvec = jnp.zeros((16,), jnp.float32)

    def zfill(r, _):
        for c in (0, 16, 32, 48, 56):
            zb[r, pl.ds(c, 16)] = zvec
        return 0
    lax.fori_loop(0, 32, zfill, 0)

    rbase = sub * RPT

    def zcopy(g, _):
        pltpu.sync_copy(zb, acc_s.at[pl.ds(rbase + 32 * g, 32)])
        return 0
    lax.fori_loop(0, RPT // 32, zcopy, 0)
    plsc.subcore_barrier()

    pltpu.sync_copy(m_hbm, mv)
    pltpu.sync_copy(ad_hbm, adt)
    mvec = mv[...]
    it = lax.broadcasted_iota(jnp.int32, (16,), 0)
    it4 = it // 4
    it3 = it & 3
    ma = jnp.take_along_axis(mvec, it3, axis=0)
    mb4 = jnp.take_along_axis(mvec, it3 + 4, axis=0)

    pltpu.sync_copy(src_hbm.at[pl.ds(wid * NBLK, NBLK)], sidx)

    def issue_gather(g, buf, buf3):
        pltpu.async_copy(dst_hbm.at[wid * NBLK + g], didx.at[buf3], gsem)
        pltpu.async_copy(hs_hbm.at[sidx.at[g, pl.ds(0, EB // 2)]],
                         hsr.at[buf, pl.ds(0, EB // 2)], gsem)
        pltpu.async_copy(hs_hbm.at[sidx.at[g, pl.ds(EB // 2, EB // 2)]],
                         hsr.at[buf, pl.ds(EB // 2, EB // 2)], gsem)

    def wait_gather(buf, buf3):
        pltpu.make_async_copy(dst_hbm.at[0], didx.at[buf3], gsem).wait()
        pltpu.make_async_copy(hs_hbm.at[sidx.at[0, pl.ds(0, EB // 2)]],
                              hsr.at[buf, pl.ds(0, EB // 2)], gsem).wait()
        pltpu.make_async_copy(hs_hbm.at[sidx.at[0, pl.ds(0, EB // 2)]],
                              hsr.at[buf, pl.ds(EB // 2, EB // 2)],
                              gsem).wait()

    def wait_scatter(buf):
        pltpu.make_async_copy(acc_hbm.at[0, pl.ds(0, EB)], mb.at[buf],
                              ssem).wait()

    issue_gather(0, 0, 0)

    def blk(g, _):
        cur = lax.rem(g, 2)
        nxt = lax.rem(g + 1, 2)
        cur3 = lax.rem(g, 3)
        nxt3 = lax.rem(g + 1, 3)
        wait_gather(cur, cur3)

        @pl.when(g >= 2)
        def _():
            wait_scatter(cur)

        @pl.when(g + 1 < NBLK)
        def _():
            issue_gather(g + 1, nxt, nxt3)

        bi = it - it + cur

        def grp(gi, _):
            dvec = didx[cur3, pl.ds(16 * gi, 16)]
            for m in range(4):
                rows = 16 * gi + 4 * m + it4
                asl = plsc.load_gather(hsr, [bi, rows, 32 + it3])
                ash = plsc.load_gather(hsr, [bi, rows, 36 + it3])
                drow = jnp.take_along_axis(dvec, 4 * m + it4, axis=0)
                wv = plsc.load_gather(
                    adt, [drow >> 1, (drow & 1) * 4 + it3])
                adl, adh = plsc.unpack(plsc.bitcast(wv, jnp.bfloat16),
                                       format=plsc.PackFormat.INTERLEAVED)
                sa = asl + adl
                sb = ash + adh
                pa = jnp.exp(jnp.maximum(sa, 0.2 * sa) - ma)
                pb = jnp.exp(jnp.maximum(sb, 0.2 * sb) - mb4)
                plsc.store_scatter(mb, [bi, rows, D + it3], pa)
                plsc.store_scatter(mb, [bi, rows, D + 4 + it3], pb)
                for ii in range(4):
                    i = 16 * gi + 4 * m + ii
                    for j in range(2):
                        w = hsr[cur, i, pl.ds(16 * j, 16)]
                        hv = plsc.bitcast(w, jnp.bfloat16)
                        av, bv = plsc.unpack(
                            hv, format=plsc.PackFormat.INTERLEAVED)
                        pj = jnp.take_along_axis(pa if j == 0 else pb,
                                                 4 * ii + it4, axis=0)
                        mb[cur, i, pl.ds(32 * j, 16)] = av * pj
                        mb[cur, i, pl.ds(32 * j + 16, 16)] = bv * pj
            return 0
        lax.fori_loop(0, EB // 16, grp, 0)

        pltpu.async_copy(mb.at[cur], acc_s.at[didx.at[cur3]], ssem, add=True)
        return 0
    lax.fori_loop(0, NBLK, blk, 0)

    wait_scatter(0)
    wait_scatter(1)
    plsc.subcore_barrier()

    pltpu.sync_copy(acc_s.at[pl.ds(rbase, RPT)],
                    acc_hbm.at[core, pl.ds(rbase, RPT)])


def _sc_edge_pass(hs, ad16, m16, src2d, dst2d):
    mesh = plsc.VectorSubcoreMesh(core_axis_name="c", subcore_axis_name="s",
                                  num_cores=2, num_subcores=16)
    f = functools.partial(
        pl.kernel,
        out_type=jax.ShapeDtypeStruct((2, NP, AW), jnp.float32),
        mesh=mesh,
        compiler_params=pltpu.CompilerParams(
            use_tc_tiling_on_sc=False, needs_layout_passes=False),
        scratch_types=[
            pltpu.VMEM((NBLK, EB), jnp.int32),
            pltpu.VMEM((3, EB), jnp.int32),
            pltpu.VMEM((NP // 2, 8), jnp.float32),
            pltpu.VMEM((2, EB, HSW), jnp.float32),
            pltpu.VMEM((2, EB, AW), jnp.float32),
            pltpu.VMEM((16,), jnp.float32),
            pltpu.VMEM((32, AW), jnp.float32),
            pltpu.VMEM_SHARED((NP, AW), jnp.float32),
            pltpu.SemaphoreType.DMA,
            pltpu.SemaphoreType.DMA,
        ],
    )(_sc_body)
    return f(hs, ad16, m16, src2d, dst2d)


def _attmat16(att, heads, feat):
    d = heads * feat
    rows = jnp.arange(d)
    if heads == 8:
        base = jnp.zeros((d, 8), jnp.float32).at[
            rows, rows // feat].set(att.reshape(d))
    else:
        base = att.reshape(d, 1) * jnp.ones((1, 8), jnp.float32)
    return jnp.concatenate([base, base], axis=1)


def _pack_hs(h, a16):
    hb = h.astype(jnp.bfloat16).reshape(NP, D // 2, 2)
    hpack = jax.lax.bitcast_convert_type(hb, jnp.float32)
    return jnp.concatenate([hpack, a16[:, 0:8]], axis=1)


_SIG = [0, 4, 1, 5, 2, 6, 3, 7]


def _pack_ad(a16):
    q = a16[:, jnp.array(_SIG, jnp.int32)].astype(jnp.bfloat16)
    w = jax.lax.bitcast_convert_type(q.reshape(NP, 4, 2), jnp.float32)
    return w.reshape(NP // 2, 8)


def kernel(x, edge_index, edge_weight, W1, att_src1, att_dst1, b1,
           W2, att_src2, att_dst2, b2):
    n = x.shape[0]
    xp = jnp.zeros((NP, IN_CH), jnp.float32).at[:n].set(x)
    loop = jnp.arange(n, dtype=edge_index.dtype)
    npad = EP - edge_index.shape[1] - n
    padv = jnp.full((npad,), n, edge_index.dtype)
    src2d = jnp.concatenate([edge_index[0], loop, padv]).reshape(-1, EB)
    dst2d = jnp.concatenate([edge_index[1], loop, padv]).reshape(-1, EB)

    perm = jnp.array(_PERM, jnp.int32)
    ams1 = _attmat16(att_src1, 8, 8)
    amd1 = _attmat16(att_dst1, 8, 8)
    ams2 = _attmat16(att_src2, 1, 64)
    amd2 = _attmat16(att_dst2, 1, 64)
    rep8p = jnp.zeros((8, D), jnp.float32).at[perm // 8, jnp.arange(D)].set(1.0)
    up = jnp.zeros((D, D), jnp.float32).at[jnp.arange(D), perm].set(1.0)
    b1p = b1[perm].reshape(1, D)
    b2p = b2[perm].reshape(1, D)
    w2p = W2[perm, :]

    h1, as1, ad1, m1 = _tc_pre(xp, W1, ams1, amd1)
    acc1 = _sc_edge_pass(_pack_hs(h1, as1), _pack_ad(ad1), m1.reshape(16),
                         src2d, dst2d)
    h2, as2, ad2, m2 = _tc_mid(acc1[0], acc1[1], b1p, rep8p, w2p, ams2, amd2)
    acc2 = _sc_edge_pass(_pack_hs(h2, as2), _pack_ad(ad2), m2.reshape(16),
                         src2d, dst2d)
    out = _tc_post(acc2[0], acc2[1], b2p, up)
    return out[:n]


# speedup vs baseline: 1.0219x; 1.0219x over previous
import functools

import jax
import jax.numpy as jnp
from jax import lax
from jax.experimental import pallas as pl
from jax.experimental.pallas import tpu as pltpu
from jax.experimental.pallas import tpu_sc as plsc

N_NODES = 10000
IN_CH = 128
D = 64
AW = 72
HSW = 40
NP = 10240
EB = 128
NBLK = 82
WPE = EB * NBLK
NW = 32
EP = WPE * NW
RPT = NP // 16

_PERM = [32 * j + 2 * m + o for j in (0, 1) for o in (0, 1) for m in range(16)]


def _leaky(v):
    return jnp.maximum(v, 0.2 * v)


def _tc_pre_body(x_ref, w_ref, ams_ref, amd_ref, h_ref, as_ref, ad_ref, m_ref):
    h = jnp.dot(x_ref[...], w_ref[...], preferred_element_type=jnp.float32)
    h_ref[...] = h
    a_s = jnp.dot(h, ams_ref[...], preferred_element_type=jnp.float32)
    a_d = jnp.dot(h, amd_ref[...], preferred_element_type=jnp.float32)
    as_ref[...] = a_s
    ad_ref[...] = a_d
    m_ref[...] = _leaky(a_s.max(axis=0) + a_d.max(axis=0)).reshape(1, 16)


def _tc_mid_body(a0_ref, a1_ref, b_ref, rep_ref, w_ref,
                 ams_ref, amd_ref, h_ref, as_ref, ad_ref, m_ref):
    s = a0_ref[:, 0:D] + a1_ref[:, 0:D]
    dp = a0_ref[:, D:D + 8] + a1_ref[:, D:D + 8]
    d64 = jnp.dot(dp, rep_ref[...], preferred_element_type=jnp.float32) + 1e-16
    x2 = s / d64 + b_ref[...]
    x2 = jnp.where(x2 > 0, x2, jnp.exp(jnp.minimum(x2, 0.0)) - 1.0)
    h = jnp.dot(x2, w_ref[...], preferred_element_type=jnp.float32)
    h_ref[...] = h
    a_s = jnp.dot(h, ams_ref[...], preferred_element_type=jnp.float32)
    a_d = jnp.dot(h, amd_ref[...], preferred_element_type=jnp.float32)
    as_ref[...] = a_s
    ad_ref[...] = a_d
    m_ref[...] = _leaky(a_s.max(axis=0) + a_d.max(axis=0)).reshape(1, 16)


def _tc_post_body(a0_ref, a1_ref, b_ref, up_ref, o_ref):
    s = a0_ref[:, 0:D] + a1_ref[:, 0:D]
    dp = a0_ref[:, D:D + 1] + a1_ref[:, D:D + 1]
    o = s / (dp + 1e-16) + b_ref[...]
    z = o - jnp.max(o, axis=1, keepdims=True)
    z = z - jnp.log(jnp.sum(jnp.exp(z), axis=1, keepdims=True))
    o_ref[...] = jnp.dot(z, up_ref[...], preferred_element_type=jnp.float32)


def _tc_pre(xp, W, ams16, amd16):
    return pl.pallas_call(
        _tc_pre_body,
        out_shape=(
            jax.ShapeDtypeStruct((NP, D), jnp.float32),
            jax.ShapeDtypeStruct((NP, 16), jnp.float32),
            jax.ShapeDtypeStruct((NP, 16), jnp.float32),
            jax.ShapeDtypeStruct((1, 16), jnp.float32),
        ),
    )(xp, W, ams16, amd16)


def _tc_mid(a0, a1, b, rep, W, ams16, amd16):
    return pl.pallas_call(
        _tc_mid_body,
        out_shape=(
            jax.ShapeDtypeStruct((NP, D), jnp.float32),
            jax.ShapeDtypeStruct((NP, 16), jnp.float32),
            jax.ShapeDtypeStruct((NP, 16), jnp.float32),
            jax.ShapeDtypeStruct((1, 16), jnp.float32),
        ),
    )(a0, a1, b, rep, W, ams16, amd16)


def _tc_post(a0, a1, b, up):
    return pl.pallas_call(
        _tc_post_body,
        out_shape=jax.ShapeDtypeStruct((NP, D), jnp.float32),
    )(a0, a1, b, up)


def _sc_body(hs_hbm, ad_hbm, m_hbm, src_hbm, dst_hbm, acc_hbm,
             sidx, didx, adt, hsr, mb, mv, zb, acc_s, gsem, ssem):
    core = lax.axis_index("c")
    sub = lax.axis_index("s")
    wid = sub * 2 + core
    zvec = jnp.zeros((16,), jnp.float32)

    def zfill(r, _):
        for c in (0, 16, 32, 48, 56):
            zb[r, pl.ds(c, 16)] = zvec
        return 0
    lax.fori_loop(0, 32, zfill, 0)

    rbase = sub * RPT

    def zcopy(g, _):
        pltpu.sync_copy(zb, acc_s.at[pl.ds(rbase + 32 * g, 32)])
        return 0
    lax.fori_loop(0, RPT // 32, zcopy, 0)
    plsc.subcore_barrier()

    pltpu.sync_copy(m_hbm, mv)
    pltpu.sync_copy(ad_hbm, adt)
    mvec = mv[...]
    it = lax.broadcasted_iota(jnp.int32, (16,), 0)
    it4 = it // 4
    it3 = it & 3
    ma = jnp.take_along_axis(mvec, it3, axis=0)
    mb4 = jnp.take_along_axis(mvec, it3 + 4, axis=0)

    pltpu.sync_copy(src_hbm.at[pl.ds(wid * NBLK, NBLK)], sidx)

    def issue_gather(g, buf, buf3):
        pltpu.async_copy(dst_hbm.at[wid * NBLK + g], didx.at[buf3], gsem)
        pltpu.async_copy(hs_hbm.at[sidx.at[g]], hsr.at[buf], gsem)

    def wait_gather(buf, buf3):
        pltpu.make_async_copy(dst_hbm.at[0], didx.at[buf3], gsem).wait()
        pltpu.make_async_copy(hs_hbm.at[sidx.at[0]], hsr.at[buf], gsem).wait()

    def wait_scatter(buf):
        pltpu.make_async_copy(acc_hbm.at[0, pl.ds(0, EB)], mb.at[buf],
                              ssem).wait()

    issue_gather(0, 0, 0)

    def blk(g, _):
        cur = lax.rem(g, 2)
        nxt = lax.rem(g + 1, 2)
        cur3 = lax.rem(g, 3)
        nxt3 = lax.rem(g + 1, 3)
        wait_gather(cur, cur3)

        @pl.when(g >= 2)
        def _():
            wait_scatter(cur)

        @pl.when(g + 1 < NBLK)
        def _():
            issue_gather(g + 1, nxt, nxt3)

        bi = it - it + cur

        def grp(gi, _):
            dvec = didx[cur3, pl.ds(16 * gi, 16)]
            for m in range(4):
                rows = 16 * gi + 4 * m + it4
                asl = plsc.load_gather(hsr, [bi, rows, 32 + it3])
                ash = plsc.load_gather(hsr, [bi, rows, 36 + it3])
                drow = jnp.take_along_axis(dvec, 4 * m + it4, axis=0)
                wv = plsc.load_gather(
                    adt, [drow >> 1, (drow & 1) * 4 + it3])
                adl, adh = plsc.unpack(plsc.bitcast(wv, jnp.bfloat16),
                                       format=plsc.PackFormat.INTERLEAVED)
                sa = asl + adl
                sb = ash + adh
                pa = jnp.exp(jnp.maximum(sa, 0.2 * sa) - ma)
                pb = jnp.exp(jnp.maximum(sb, 0.2 * sb) - mb4)
                plsc.store_scatter(mb, [bi, rows, D + it3], pa)
                plsc.store_scatter(mb, [bi, rows, D + 4 + it3], pb)
                for ii in range(4):
                    i = 16 * gi + 4 * m + ii
                    for j in range(2):
                        w = hsr[cur, i, pl.ds(16 * j, 16)]
                        hv = plsc.bitcast(w, jnp.bfloat16)
                        av, bv = plsc.unpack(
                            hv, format=plsc.PackFormat.INTERLEAVED)
                        pj = jnp.take_along_axis(pa if j == 0 else pb,
                                                 4 * ii + it4, axis=0)
                        mb[cur, i, pl.ds(32 * j, 16)] = av * pj
                        mb[cur, i, pl.ds(32 * j + 16, 16)] = bv * pj
            return 0
        lax.fori_loop(0, EB // 16, grp, 0)

        pltpu.async_copy(mb.at[cur], acc_s.at[didx.at[cur3]], ssem, add=True)
        return 0
    lax.fori_loop(0, NBLK, blk, 0)

    wait_scatter(0)
    wait_scatter(1)
    plsc.subcore_barrier()

    pltpu.sync_copy(acc_s.at[pl.ds(rbase, RPT)],
                    acc_hbm.at[core, pl.ds(rbase, RPT)])


def _sc_edge_pass(hs, ad16, m16, src2d, dst2d):
    mesh = plsc.VectorSubcoreMesh(core_axis_name="c", subcore_axis_name="s",
                                  num_cores=2, num_subcores=16)
    f = functools.partial(
        pl.kernel,
        out_type=jax.ShapeDtypeStruct((2, NP, AW), jnp.float32),
        mesh=mesh,
        compiler_params=pltpu.CompilerParams(
            use_tc_tiling_on_sc=False, needs_layout_passes=False),
        scratch_types=[
            pltpu.VMEM((NBLK, EB), jnp.int32),
            pltpu.VMEM((3, EB), jnp.int32),
            pltpu.VMEM((NP // 2, 8), jnp.float32),
            pltpu.VMEM((2, EB, HSW), jnp.float32),
            pltpu.VMEM((2, EB, AW), jnp.float32),
            pltpu.VMEM((16,), jnp.float32),
            pltpu.VMEM((32, AW), jnp.float32),
            pltpu.VMEM_SHARED((NP, AW), jnp.float32),
            pltpu.SemaphoreType.DMA,
            pltpu.SemaphoreType.DMA,
        ],
    )(_sc_body)
    return f(hs, ad16, m16, src2d, dst2d)


def _attmat16(att, heads, feat):
    d = heads * feat
    rows = jnp.arange(d)
    if heads == 8:
        base = jnp.zeros((d, 8), jnp.float32).at[
            rows, rows // feat].set(att.reshape(d))
    else:
        base = att.reshape(d, 1) * jnp.ones((1, 8), jnp.float32)
    return jnp.concatenate([base, base], axis=1)


def _pack_hs(h, a16):
    hb = h.astype(jnp.bfloat16).reshape(NP, D // 2, 2)
    hpack = jax.lax.bitcast_convert_type(hb, jnp.float32)
    return jnp.concatenate([hpack, a16[:, 0:8]], axis=1)


_SIG = [0, 4, 1, 5, 2, 6, 3, 7]


def _pack_ad(a16):
    q = a16[:, jnp.array(_SIG, jnp.int32)].astype(jnp.bfloat16)
    w = jax.lax.bitcast_convert_type(q.reshape(NP, 4, 2), jnp.float32)
    return w.reshape(NP // 2, 8)


def kernel(x, edge_index, edge_weight, W1, att_src1, att_dst1, b1,
           W2, att_src2, att_dst2, b2):
    n = x.shape[0]
    xp = jnp.zeros((NP, IN_CH), jnp.float32).at[:n].set(x)
    loop = jnp.arange(n, dtype=edge_index.dtype)
    npad = EP - edge_index.shape[1] - n
    padv = jnp.full((npad,), n, edge_index.dtype)
    src2d = jnp.concatenate([edge_index[0], loop, padv]).reshape(-1, EB)
    dst2d = jnp.concatenate([edge_index[1], loop, padv]).reshape(-1, EB)

    perm = jnp.array(_PERM, jnp.int32)
    ams1 = _attmat16(att_src1, 8, 8)
    amd1 = _attmat16(att_dst1, 8, 8)
    ams2 = _attmat16(att_src2, 1, 64)
    amd2 = _attmat16(att_dst2, 1, 64)
    rep8p = jnp.zeros((8, D), jnp.float32).at[perm // 8, jnp.arange(D)].set(1.0)
    up = jnp.zeros((D, D), jnp.float32).at[jnp.arange(D), perm].set(1.0)
    b1p = b1[perm].reshape(1, D)
    b2p = b2[perm].reshape(1, D)
    w2p = W2[perm, :]

    h1, as1, ad1, m1 = _tc_pre(xp, W1, ams1, amd1)
    acc1 = _sc_edge_pass(_pack_hs(h1, as1), _pack_ad(ad1), m1.reshape(16),
                         src2d, dst2d)
    h2, as2, ad2, m2 = _tc_mid(acc1[0], acc1[1], b1p, rep8p, w2p, ams2, amd2)
    acc2 = _sc_edge_pass(_pack_hs(h2, as2), _pack_ad(ad2), m2.reshape(16),
                         src2d, dst2d)
    out = _tc_post(acc2[0], acc2[1], b2p, up)
    return out[:n]


# speedup vs baseline: 1.1327x; 1.1085x over previous
import functools

import jax
import jax.numpy as jnp
from jax import lax
from jax.experimental import pallas as pl
from jax.experimental.pallas import tpu as pltpu
from jax.experimental.pallas import tpu_sc as plsc

N_NODES = 10000
IN_CH = 128
D = 64
AW = 72
HSW = 40
NP = 10240
EB = 128
NBLK = 82
WPE = EB * NBLK
NW = 32
EP = WPE * NW
RPT = NP // 16

_PERM = [32 * j + 2 * m + o for j in (0, 1) for o in (0, 1) for m in range(16)]


def _leaky(v):
    return jnp.maximum(v, 0.2 * v)


def _packpair(lo, hi):
    ul = jax.lax.bitcast_convert_type(lo, jnp.int32)
    uh = jax.lax.bitcast_convert_type(hi, jnp.int32)
    rl = jax.lax.shift_right_logical(ul + 0x7FFF + ((ul >> 16) & 1), 16)
    rh = (uh + 0x7FFF + ((uh >> 16) & 1)) & jnp.int32(-65536)
    return jax.lax.bitcast_convert_type(rl | rh, jnp.float32)


def _emit_tables(h, a_s, a_d):
    packed = _packpair(h[:, 0:32], h[:, 32:64])
    hs = jnp.concatenate([packed, a_s[:, 0:8]], axis=1)
    adp = _packpair(a_d[:, 0:4], a_d[:, 4:8])
    return hs, adp


def _tc_pre_body(x_ref, w_ref, ams_ref, amd_ref, hs_ref, adp_ref, m_ref):
    h = jnp.dot(x_ref[...], w_ref[...], preferred_element_type=jnp.float32)
    a_s = jnp.dot(h, ams_ref[...], preferred_element_type=jnp.float32)
    a_d = jnp.dot(h, amd_ref[...], preferred_element_type=jnp.float32)
    hs, adp = _emit_tables(h, a_s, a_d)
    zrows = NP - N_NODES
    hs_ref[...] = jnp.concatenate(
        [hs, jnp.zeros((zrows, HSW), jnp.float32)], axis=0)
    adp_ref[...] = jnp.concatenate(
        [adp, jnp.zeros((zrows, 4), jnp.float32)], axis=0)
    m_ref[...] = _leaky(a_s.max(axis=0) + a_d.max(axis=0)).reshape(1, 16)


def _tc_mid_body(a_ref, b_ref, rep_ref, w_ref, ams_ref, amd_ref,
                 hs_ref, adp_ref, m_ref):
    s = a_ref[0, :, 0:D] + a_ref[1, :, 0:D]
    dp = a_ref[0, :, D:D + 8] + a_ref[1, :, D:D + 8]
    d64 = jnp.dot(dp, rep_ref[...], preferred_element_type=jnp.float32) + 1e-16
    x2 = s / d64 + b_ref[...]
    x2 = jnp.where(x2 > 0, x2, jnp.exp(jnp.minimum(x2, 0.0)) - 1.0)
    h = jnp.dot(x2, w_ref[...], preferred_element_type=jnp.float32)
    a_s = jnp.dot(h, ams_ref[...], preferred_element_type=jnp.float32)
    a_d = jnp.dot(h, amd_ref[...], preferred_element_type=jnp.float32)
    hs, adp = _emit_tables(h, a_s, a_d)
    hs_ref[...] = hs
    adp_ref[...] = adp
    m_ref[...] = _leaky(a_s.max(axis=0) + a_d.max(axis=0)).reshape(1, 16)


def _tc_post_body(a_ref, b_ref, up_ref, o_ref):
    s = a_ref[0, :, 0:D] + a_ref[1, :, 0:D]
    dp = a_ref[0, :, D:D + 1] + a_ref[1, :, D:D + 1]
    o = s / (dp + 1e-16) + b_ref[...]
    z = o - jnp.max(o, axis=1, keepdims=True)
    z = z - jnp.log(jnp.sum(jnp.exp(z), axis=1, keepdims=True))
    res = jnp.dot(z, up_ref[...], preferred_element_type=jnp.float32)
    o_ref[...] = res[0:N_NODES, :]


def _tc_pre(x, W, ams16, amd16):
    return pl.pallas_call(
        _tc_pre_body,
        out_shape=(
            jax.ShapeDtypeStruct((NP, HSW), jnp.float32),
            jax.ShapeDtypeStruct((NP, 4), jnp.float32),
            jax.ShapeDtypeStruct((1, 16), jnp.float32),
        ),
    )(x, W, ams16, amd16)


def _tc_mid(acc, b, rep, W, ams16, amd16):
    return pl.pallas_call(
        _tc_mid_body,
        out_shape=(
            jax.ShapeDtypeStruct((NP, HSW), jnp.float32),
            jax.ShapeDtypeStruct((NP, 4), jnp.float32),
            jax.ShapeDtypeStruct((1, 16), jnp.float32),
        ),
    )(acc, b, rep, W, ams16, amd16)


def _tc_post(acc, b, up):
    return pl.pallas_call(
        _tc_post_body,
        out_shape=jax.ShapeDtypeStruct((N_NODES, D), jnp.float32),
    )(acc, b, up)


def _sc_body(hs_hbm, ad_hbm, m_hbm, src_hbm, dst_hbm, acc_hbm,
             sidx, didx, adt, hsr, mb, mv, zb, acc_s, gsem, ssem):
    core = lax.axis_index("c")
    sub = lax.axis_index("s")
    wid = sub * 2 + core
    zvec = jnp.zeros((16,), jnp.float32)

    def zfill(r, _):
        for c in (0, 16, 32, 48, 56):
            zb[r, pl.ds(c, 16)] = zvec
        return 0
    lax.fori_loop(0, 32, zfill, 0)

    rbase = sub * RPT

    def zcopy(g, _):
        pltpu.sync_copy(zb, acc_s.at[pl.ds(rbase + 32 * g, 32)])
        return 0
    lax.fori_loop(0, RPT // 32, zcopy, 0)
    plsc.subcore_barrier()

    pltpu.sync_copy(m_hbm, mv)
    pltpu.sync_copy(ad_hbm, adt)
    mvec = mv[...]
    it = lax.broadcasted_iota(jnp.int32, (16,), 0)
    it4 = it // 4
    it3 = it & 3
    ma = jnp.take_along_axis(mvec, it3, axis=0)
    mb4 = jnp.take_along_axis(mvec, it3 + 4, axis=0)

    pltpu.sync_copy(src_hbm.at[pl.ds(wid * NBLK, NBLK)], sidx)

    def issue_gather(g, buf, buf3):
        pltpu.async_copy(dst_hbm.at[wid * NBLK + g], didx.at[buf3], gsem)
        pltpu.async_copy(hs_hbm.at[sidx.at[g]], hsr.at[buf], gsem)

    def wait_gather(buf, buf3):
        pltpu.make_async_copy(dst_hbm.at[0], didx.at[buf3], gsem).wait()
        pltpu.make_async_copy(hs_hbm.at[sidx.at[0]], hsr.at[buf], gsem).wait()

    def wait_scatter(buf):
        pltpu.make_async_copy(acc_hbm.at[0, pl.ds(0, EB)], mb.at[buf],
                              ssem).wait()

    issue_gather(0, 0, 0)

    def blk(g, _):
        cur = lax.rem(g, 2)
        nxt = lax.rem(g + 1, 2)
        cur3 = lax.rem(g, 3)
        nxt3 = lax.rem(g + 1, 3)
        wait_gather(cur, cur3)

        @pl.when(g >= 2)
        def _():
            wait_scatter(cur)

        @pl.when(g + 1 < NBLK)
        def _():
            issue_gather(g + 1, nxt, nxt3)

        bi = it - it + cur

        def grp(gi, _):
            dvec = didx[cur3, pl.ds(16 * gi, 16)]
            for m in range(4):
                rows = 16 * gi + 4 * m + it4
                asl = plsc.load_gather(hsr, [bi, rows, 32 + it3])
                ash = plsc.load_gather(hsr, [bi, rows, 36 + it3])
                drow = jnp.take_along_axis(dvec, 4 * m + it4, axis=0)
                wv = plsc.load_gather(
                    adt, [drow >> 1, (drow & 1) * 4 + it3])
                adl, adh = plsc.unpack(plsc.bitcast(wv, jnp.bfloat16),
                                       format=plsc.PackFormat.INTERLEAVED)
                sa = asl + adl
                sb = ash + adh
                pa = jnp.exp(jnp.maximum(sa, 0.2 * sa) - ma)
                pb = jnp.exp(jnp.maximum(sb, 0.2 * sb) - mb4)
                plsc.store_scatter(mb, [bi, rows, D + it3], pa)
                plsc.store_scatter(mb, [bi, rows, D + 4 + it3], pb)
                for ii in range(4):
                    i = 16 * gi + 4 * m + ii
                    for j in range(2):
                        w = hsr[cur, i, pl.ds(16 * j, 16)]
                        hv = plsc.bitcast(w, jnp.bfloat16)
                        av, bv = plsc.unpack(
                            hv, format=plsc.PackFormat.INTERLEAVED)
                        pj = jnp.take_along_axis(pa if j == 0 else pb,
                                                 4 * ii + it4, axis=0)
                        mb[cur, i, pl.ds(32 * j, 16)] = av * pj
                        mb[cur, i, pl.ds(32 * j + 16, 16)] = bv * pj
            return 0
        lax.fori_loop(0, EB // 16, grp, 0)

        pltpu.async_copy(mb.at[cur], acc_s.at[didx.at[cur3]], ssem, add=True)
        return 0
    lax.fori_loop(0, NBLK, blk, 0)

    wait_scatter(0)
    wait_scatter(1)
    plsc.subcore_barrier()

    pltpu.sync_copy(acc_s.at[pl.ds(rbase, RPT)],
                    acc_hbm.at[core, pl.ds(rbase, RPT)])


def _sc_edge_pass(hs, ad16, m16, src2d, dst2d):
    mesh = plsc.VectorSubcoreMesh(core_axis_name="c", subcore_axis_name="s",
                                  num_cores=2, num_subcores=16)
    f = functools.partial(
        pl.kernel,
        out_type=jax.ShapeDtypeStruct((2, NP, AW), jnp.float32),
        mesh=mesh,
        compiler_params=pltpu.CompilerParams(
            use_tc_tiling_on_sc=False, needs_layout_passes=False),
        scratch_types=[
            pltpu.VMEM((NBLK, EB), jnp.int32),
            pltpu.VMEM((3, EB), jnp.int32),
            pltpu.VMEM((NP // 2, 8), jnp.float32),
            pltpu.VMEM((2, EB, HSW), jnp.float32),
            pltpu.VMEM((2, EB, AW), jnp.float32),
            pltpu.VMEM((16,), jnp.float32),
            pltpu.VMEM((32, AW), jnp.float32),
            pltpu.VMEM_SHARED((NP, AW), jnp.float32),
            pltpu.SemaphoreType.DMA,
            pltpu.SemaphoreType.DMA,
        ],
    )(_sc_body)
    return f(hs, ad16, m16, src2d, dst2d)


def _attmat16(att, heads, feat):
    d = heads * feat
    rows = jnp.arange(d)
    if heads == 8:
        base = jnp.zeros((d, 8), jnp.float32).at[
            rows, rows // feat].set(att.reshape(d))
    else:
        base = att.reshape(d, 1) * jnp.ones((1, 8), jnp.float32)
    return jnp.concatenate([base, base], axis=1)


_EPERM = [2 * m for m in range(32)] + [2 * m + 1 for m in range(32)]


def kernel(x, edge_index, edge_weight, W1, att_src1, att_dst1, b1,
           W2, att_src2, att_dst2, b2):
    n = x.shape[0]
    loop = jnp.arange(n, dtype=edge_index.dtype)
    npad = EP - edge_index.shape[1] - n
    padv = jnp.full((npad,), n, edge_index.dtype)
    src2d = jnp.concatenate([edge_index[0], loop, padv]).reshape(-1, EB)
    dst2d = jnp.concatenate([edge_index[1], loop, padv]).reshape(-1, EB)

    perm = jnp.array(_PERM, jnp.int32)
    eperm = jnp.array(_EPERM, jnp.int32)
    w1e = W1[:, eperm]
    ams1 = _attmat16(att_src1, 8, 8)[eperm, :]
    amd1 = _attmat16(att_dst1, 8, 8)[eperm, :]
    ams2 = _attmat16(att_src2, 1, 64)[eperm, :]
    amd2 = _attmat16(att_dst2, 1, 64)[eperm, :]
    rep8p = jnp.zeros((8, D), jnp.float32).at[perm // 8, jnp.arange(D)].set(1.0)
    up = jnp.zeros((D, D), jnp.float32).at[jnp.arange(D), perm].set(1.0)
    b1p = b1[perm].reshape(1, D)
    b2p = b2[perm].reshape(1, D)
    w2pe = W2[perm, :][:, eperm]

    hs1, adp1, m1 = _tc_pre(x, w1e, ams1, amd1)
    acc1 = _sc_edge_pass(hs1, adp1.reshape(NP // 2, 8), m1.reshape(16),
                         src2d, dst2d)
    hs2, adp2, m2 = _tc_mid(acc1, b1p, rep8p, w2pe, ams2, amd2)
    acc2 = _sc_edge_pass(hs2, adp2.reshape(NP // 2, 8), m2.reshape(16),
                         src2d, dst2d)
    return _tc_post(acc2, b2p, up)


# speedup vs baseline: 1.1492x; 1.0145x over previous
import functools

import jax
import jax.numpy as jnp
from jax import lax
from jax.experimental import pallas as pl
from jax.experimental.pallas import tpu as pltpu
from jax.experimental.pallas import tpu_sc as plsc

N_NODES = 10000
IN_CH = 128
D = 64
AW = 72
HSW = 40
NP = 10240
EB = 128
NBLK = 82
WPE = EB * NBLK
NW = 32
EP = WPE * NW
RPT = NP // 16

_PERM = [32 * j + 2 * m + o for j in (0, 1) for o in (0, 1) for m in range(16)]


def _leaky(v):
    return jnp.maximum(v, 0.2 * v)


def _packpair(lo, hi):
    ul = jax.lax.bitcast_convert_type(lo, jnp.int32)
    uh = jax.lax.bitcast_convert_type(hi, jnp.int32)
    rl = jax.lax.shift_right_logical(ul + 0x7FFF + ((ul >> 16) & 1), 16)
    rh = (uh + 0x7FFF + ((uh >> 16) & 1)) & jnp.int32(-65536)
    return jax.lax.bitcast_convert_type(rl | rh, jnp.float32)


def _emit_tables(h, a_s, a_d):
    packed = _packpair(h[:, 0:32], h[:, 32:64])
    hs = jnp.concatenate([packed, a_s[:, 0:8]], axis=1)
    adp = _packpair(a_d[:, 0:4], a_d[:, 4:8])
    return hs, adp


def _tc_pre_body(x_ref, w_ref, ams_ref, amd_ref, hs_ref, adp_ref, m_ref):
    h = jnp.dot(x_ref[...], w_ref[...], preferred_element_type=jnp.float32)
    a_s = jnp.dot(h, ams_ref[...], preferred_element_type=jnp.float32)
    a_d = jnp.dot(h, amd_ref[...], preferred_element_type=jnp.float32)
    hs, adp = _emit_tables(h, a_s, a_d)
    zrows = NP - N_NODES
    hs_ref[...] = jnp.concatenate(
        [hs, jnp.zeros((zrows, HSW), jnp.float32)], axis=0)
    adp_ref[...] = jnp.concatenate(
        [adp, jnp.zeros((zrows, 4), jnp.float32)], axis=0)
    m_ref[...] = _leaky(a_s.max(axis=0) + a_d.max(axis=0)).reshape(1, 16)


def _tc_mid_body(a_ref, b_ref, rep_ref, w_ref, ams_ref, amd_ref,
                 hs_ref, adp_ref, m_ref):
    s = a_ref[0, :, 0:D] + a_ref[1, :, 0:D]
    dp = a_ref[0, :, D:D + 8] + a_ref[1, :, D:D + 8]
    d64 = jnp.dot(dp, rep_ref[...], preferred_element_type=jnp.float32) + 1e-16
    x2 = s / d64 + b_ref[...]
    x2 = jnp.where(x2 > 0, x2, jnp.exp(jnp.minimum(x2, 0.0)) - 1.0)
    h = jnp.dot(x2, w_ref[...], preferred_element_type=jnp.float32)
    a_s = jnp.dot(h, ams_ref[...], preferred_element_type=jnp.float32)
    a_d = jnp.dot(h, amd_ref[...], preferred_element_type=jnp.float32)
    hs, adp = _emit_tables(h, a_s, a_d)
    hs_ref[...] = hs
    adp_ref[...] = adp
    m_ref[...] = _leaky(a_s.max(axis=0) + a_d.max(axis=0)).reshape(1, 16)


def _tc_post_body(a_ref, b_ref, up_ref, o_ref):
    s = a_ref[0, :, 0:D] + a_ref[1, :, 0:D]
    dp = a_ref[0, :, D:D + 1] + a_ref[1, :, D:D + 1]
    o = s / (dp + 1e-16) + b_ref[...]
    z = o - jnp.max(o, axis=1, keepdims=True)
    z = z - jnp.log(jnp.sum(jnp.exp(z), axis=1, keepdims=True))
    res = jnp.dot(z, up_ref[...], preferred_element_type=jnp.float32)
    o_ref[...] = res[0:N_NODES, :]


def _tc_pre(x, W, ams16, amd16):
    return pl.pallas_call(
        _tc_pre_body,
        out_shape=(
            jax.ShapeDtypeStruct((NP, HSW), jnp.float32),
            jax.ShapeDtypeStruct((NP, 4), jnp.float32),
            jax.ShapeDtypeStruct((1, 16), jnp.float32),
        ),
    )(x, W, ams16, amd16)


def _tc_mid(acc, b, rep, W, ams16, amd16):
    return pl.pallas_call(
        _tc_mid_body,
        out_shape=(
            jax.ShapeDtypeStruct((NP, HSW), jnp.float32),
            jax.ShapeDtypeStruct((NP, 4), jnp.float32),
            jax.ShapeDtypeStruct((1, 16), jnp.float32),
        ),
    )(acc, b, rep, W, ams16, amd16)


def _tc_post(acc, b, up):
    return pl.pallas_call(
        _tc_post_body,
        out_shape=jax.ShapeDtypeStruct((N_NODES, D), jnp.float32),
    )(acc, b, up)


def _sc_body(hs_hbm, ad_hbm, m_hbm, src_hbm, dst_hbm, acc_hbm,
             sidx, didx, adt, hsr, mb, mv, zb, acc_s, gsem, ssem):
    core = lax.axis_index("c")
    sub = lax.axis_index("s")
    wid = sub * 2 + core
    zvec = jnp.zeros((16,), jnp.float32)

    pltpu.async_copy(m_hbm, mv, ssem)
    pltpu.async_copy(ad_hbm, adt, ssem)
    pltpu.async_copy(src_hbm.at[pl.ds(wid * NBLK, NBLK)], sidx, ssem)

    def zfill(r, _):
        for c in (0, 16, 32, 48, 56):
            zb[r, pl.ds(c, 16)] = zvec
        return 0
    lax.fori_loop(0, 32, zfill, 0)

    rbase = sub * RPT

    def zcopy(g, _):
        pltpu.async_copy(zb, acc_s.at[pl.ds(rbase + 32 * g, 32)], gsem)
        return 0
    lax.fori_loop(0, RPT // 32, zcopy, 0)

    def zdrain(g, _):
        pltpu.make_async_copy(zb, acc_s.at[pl.ds(rbase, 32)], gsem).wait()
        return 0
    lax.fori_loop(0, RPT // 32, zdrain, 0)
    pltpu.make_async_copy(m_hbm, mv, ssem).wait()
    pltpu.make_async_copy(ad_hbm, adt, ssem).wait()
    pltpu.make_async_copy(src_hbm.at[pl.ds(0, NBLK)], sidx, ssem).wait()
    plsc.subcore_barrier()

    mvec = mv[...]
    it = lax.broadcasted_iota(jnp.int32, (16,), 0)
    it4 = it // 4
    it3 = it & 3
    ma = jnp.take_along_axis(mvec, it3, axis=0)
    mb4 = jnp.take_along_axis(mvec, it3 + 4, axis=0)

    def issue_gather(g, buf, buf3):
        pltpu.async_copy(dst_hbm.at[wid * NBLK + g], didx.at[buf3], gsem)
        pltpu.async_copy(hs_hbm.at[sidx.at[g]], hsr.at[buf], gsem)

    def wait_gather(buf, buf3):
        pltpu.make_async_copy(dst_hbm.at[0], didx.at[buf3], gsem).wait()
        pltpu.make_async_copy(hs_hbm.at[sidx.at[0]], hsr.at[buf], gsem).wait()

    def wait_scatter(buf):
        pltpu.make_async_copy(acc_hbm.at[0, pl.ds(0, EB)], mb.at[buf],
                              ssem).wait()

    issue_gather(0, 0, 0)

    def blk(g, _):
        cur = lax.rem(g, 2)
        nxt = lax.rem(g + 1, 2)
        cur3 = lax.rem(g, 3)
        nxt3 = lax.rem(g + 1, 3)
        wait_gather(cur, cur3)

        @pl.when(g >= 2)
        def _():
            wait_scatter(cur)

        @pl.when(g + 1 < NBLK)
        def _():
            issue_gather(g + 1, nxt, nxt3)

        bi = it - it + cur

        def grp(gi, _):
            dvec = didx[cur3, pl.ds(16 * gi, 16)]
            for m in range(4):
                rows = 16 * gi + 4 * m + it4
                asl = plsc.load_gather(hsr, [bi, rows, 32 + it3])
                ash = plsc.load_gather(hsr, [bi, rows, 36 + it3])
                drow = jnp.take_along_axis(dvec, 4 * m + it4, axis=0)
                wv = plsc.load_gather(
                    adt, [drow >> 1, (drow & 1) * 4 + it3])
                adl, adh = plsc.unpack(plsc.bitcast(wv, jnp.bfloat16),
                                       format=plsc.PackFormat.INTERLEAVED)
                sa = asl + adl
                sb = ash + adh
                pa = jnp.exp(jnp.maximum(sa, 0.2 * sa) - ma)
                pb = jnp.exp(jnp.maximum(sb, 0.2 * sb) - mb4)
                plsc.store_scatter(mb, [bi, rows, D + it3], pa)
                plsc.store_scatter(mb, [bi, rows, D + 4 + it3], pb)
                for ii in range(4):
                    i = 16 * gi + 4 * m + ii
                    for j in range(2):
                        w = hsr[cur, i, pl.ds(16 * j, 16)]
                        hv = plsc.bitcast(w, jnp.bfloat16)
                        av, bv = plsc.unpack(
                            hv, format=plsc.PackFormat.INTERLEAVED)
                        pj = jnp.take_along_axis(pa if j == 0 else pb,
                                                 4 * ii + it4, axis=0)
                        mb[cur, i, pl.ds(32 * j, 16)] = av * pj
                        mb[cur, i, pl.ds(32 * j + 16, 16)] = bv * pj
            return 0
        lax.fori_loop(0, EB // 16, grp, 0)

        pltpu.async_copy(mb.at[cur], acc_s.at[didx.at[cur3]], ssem, add=True)
        return 0
    lax.fori_loop(0, NBLK, blk, 0)

    wait_scatter(0)
    wait_scatter(1)
    plsc.subcore_barrier()

    pltpu.sync_copy(acc_s.at[pl.ds(rbase, RPT)],
                    acc_hbm.at[core, pl.ds(rbase, RPT)])


def _sc_edge_pass(hs, ad16, m16, src2d, dst2d):
    mesh = plsc.VectorSubcoreMesh(core_axis_name="c", subcore_axis_name="s",
                                  num_cores=2, num_subcores=16)
    f = functools.partial(
        pl.kernel,
        out_type=jax.ShapeDtypeStruct((2, NP, AW), jnp.float32),
        mesh=mesh,
        compiler_params=pltpu.CompilerParams(
            use_tc_tiling_on_sc=False, needs_layout_passes=False),
        scratch_types=[
            pltpu.VMEM((NBLK, EB), jnp.int32),
            pltpu.VMEM((3, EB), jnp.int32),
            pltpu.VMEM((NP // 2, 8), jnp.float32),
            pltpu.VMEM((2, EB, HSW), jnp.float32),
            pltpu.VMEM((2, EB, AW), jnp.float32),
            pltpu.VMEM((16,), jnp.float32),
            pltpu.VMEM((32, AW), jnp.float32),
            pltpu.VMEM_SHARED((NP, AW), jnp.float32),
            pltpu.SemaphoreType.DMA,
            pltpu.SemaphoreType.DMA,
        ],
    )(_sc_body)
    return f(hs, ad16, m16, src2d, dst2d)


def _attmat16(att, heads, feat):
    d = heads * feat
    rows = jnp.arange(d)
    if heads == 8:
        base = jnp.zeros((d, 8), jnp.float32).at[
            rows, rows // feat].set(att.reshape(d))
    else:
        base = att.reshape(d, 1) * jnp.ones((1, 8), jnp.float32)
    return jnp.concatenate([base, base], axis=1)


_EPERM = [2 * m for m in range(32)] + [2 * m + 1 for m in range(32)]


def kernel(x, edge_index, edge_weight, W1, att_src1, att_dst1, b1,
           W2, att_src2, att_dst2, b2):
    n = x.shape[0]
    loop = jnp.arange(n, dtype=edge_index.dtype)
    npad = EP - edge_index.shape[1] - n
    padv = jnp.full((npad,), n, edge_index.dtype)
    src2d = jnp.concatenate([edge_index[0], loop, padv]).reshape(-1, EB)
    dst2d = jnp.concatenate([edge_index[1], loop, padv]).reshape(-1, EB)

    perm = jnp.array(_PERM, jnp.int32)
    eperm = jnp.array(_EPERM, jnp.int32)
    w1e = W1[:, eperm]
    ams1 = _attmat16(att_src1, 8, 8)[eperm, :]
    amd1 = _attmat16(att_dst1, 8, 8)[eperm, :]
    ams2 = _attmat16(att_src2, 1, 64)[eperm, :]
    amd2 = _attmat16(att_dst2, 1, 64)[eperm, :]
    rep8p = jnp.zeros((8, D), jnp.float32).at[perm // 8, jnp.arange(D)].set(1.0)
    up = jnp.zeros((D, D), jnp.float32).at[jnp.arange(D), perm].set(1.0)
    b1p = b1[perm].reshape(1, D)
    b2p = b2[perm].reshape(1, D)
    w2pe = W2[perm, :][:, eperm]

    hs1, adp1, m1 = _tc_pre(x, w1e, ams1, amd1)
    acc1 = _sc_edge_pass(hs1, adp1.reshape(NP // 2, 8), m1.reshape(16),
                         src2d, dst2d)
    hs2, adp2, m2 = _tc_mid(acc1, b1p, rep8p, w2pe, ams2, amd2)
    acc2 = _sc_edge_pass(hs2, adp2.reshape(NP // 2, 8), m2.reshape(16),
                         src2d, dst2d)
    return _tc_post(acc2, b2p, up)
